# Initial kernel scaffold; baseline (speedup 1.0000x reference)
import jax
import jax.numpy as jnp
from jax.experimental import pallas as pl


def kernel(x, sl_pos, sl_neg, kg_ppi, kg_reactome, kg_corum, kg_go_f, kg_go_c, kg_go_p, kg_kegg, params):
    raise NotImplementedError("write your pallas kernel here")


# speedup vs baseline: 4.4618x; 4.4618x over previous
import functools

import jax
import jax.numpy as jnp
from jax.experimental import pallas as pl
from jax.experimental.pallas import tpu as pltpu

USED_GRAPHS = ["sl", "ppi", "reactome", "go_f", "go_c", "go_p", "kegg"]
NG = len(USED_GRAPHS)
DEC_BLOCK = 1024


def _decode_mlp_body(emb_ref, wg_ref, bg_ref, ex_ref, l1w_ref, l1b_ref,
                     l2w_ref, l2b_ref, l3w_ref, l3b_ref, out_ref):
    emb = emb_ref[...]
    wv = jnp.dot(emb, wg_ref[...], preferred_element_type=jnp.float32) + bg_ref[...]
    wvx = jnp.dot(wv, ex_ref[...], preferred_element_type=jnp.float32)
    feats = emb * wvx
    h = jnp.maximum(jnp.dot(feats, l1w_ref[...], preferred_element_type=jnp.float32)
                    + l1b_ref[...], 0.0)
    h = jnp.maximum(jnp.dot(h, l2w_ref[...], preferred_element_type=jnp.float32)
                    + l2b_ref[...], 0.0)
    o = jnp.dot(h, l3w_ref[...], preferred_element_type=jnp.float32) + l3b_ref[...]
    out_ref[...] = 1.0 / (1.0 + jnp.exp(-o))


def _decode_mlp(emb, params):
    ne = emb.shape[0]
    d = NG * 16
    wg = jnp.zeros((d, NG), jnp.float32)
    for i, g in enumerate(USED_GRAPHS):
        wg = wg.at[16 * i:16 * (i + 1), i].set(params["w_" + g + "_W"][:, 0])
    bg = jnp.stack([params["w_" + g + "_b"][0] for g in USED_GRAPHS])[None, :]
    ex = jnp.repeat(jnp.eye(NG, dtype=jnp.float32), 16, axis=1)

    grid = ne // DEC_BLOCK
    full = lambda shape: pl.BlockSpec(shape, lambda i: (0, 0))
    out = pl.pallas_call(
        _decode_mlp_body,
        grid=(grid,),
        in_specs=[
            pl.BlockSpec((DEC_BLOCK, d), lambda i: (i, 0)),
            full(wg.shape), full(bg.shape), full(ex.shape),
            full(params["L1_W"].shape), full((1, 32)),
            full(params["L2_W"].shape), full((1, 16)),
            full(params["L3_W"].shape), full((1, 1)),
        ],
        out_specs=pl.BlockSpec((DEC_BLOCK, 1), lambda i: (i, 0)),
        out_shape=jax.ShapeDtypeStruct((ne, 1), jnp.float32),
    )(emb, wg, bg, ex,
      params["L1_W"], params["L1_b"][None, :],
      params["L2_W"], params["L2_b"][None, :],
      params["L3_W"], params["L3_b"][None, :])
    return out[:, 0]


def _encoder(x, ei, params, g):
    n = x.shape[0]
    src, dst = ei[0], ei[1]
    deg = jax.ops.segment_sum(jnp.ones(src.shape, jnp.float32), dst,
                              num_segments=n) + 1.0
    dinv = deg ** -0.5
    t = dinv[:, None] * (x @ params[g + "_W1"])
    mp = jax.ops.segment_sum(t[src], dst, num_segments=n) + t
    h = jax.nn.relu(dinv[:, None] * mp + params[g + "_b1"])
    t2 = dinv[:, None] * (h @ params[g + "_W2"])
    mp2 = jax.ops.segment_sum(t2[src], dst, num_segments=n) + t2
    return dinv[:, None] * mp2 + params[g + "_b2"]


def kernel(x, sl_pos, sl_neg, kg_ppi, kg_reactome, kg_corum, kg_go_f,
           kg_go_c, kg_go_p, kg_kegg, params):
    edge_map = {"sl": sl_pos, "ppi": kg_ppi, "reactome": kg_reactome,
                "go_f": kg_go_f, "go_c": kg_go_c, "go_p": kg_go_p,
                "kegg": kg_kegg}
    zs = [_encoder(x, edge_map[g], params, g) for g in USED_GRAPHS]
    zcat = jnp.concatenate(zs, axis=1)
    ue = jnp.concatenate([sl_pos[0], sl_neg[0]])
    ve = jnp.concatenate([sl_pos[1], sl_neg[1]])
    emb = zcat[ue] + zcat[ve]
    return _decode_mlp(emb, params)


# speedup vs baseline: 10.2305x; 2.2929x over previous
import functools

import jax
import jax.numpy as jnp
from jax import lax
from jax.experimental import pallas as pl
from jax.experimental.pallas import tpu as pltpu
from jax.experimental.pallas import tpu_sc as plsc

USED_GRAPHS = ["sl", "ppi", "reactome", "go_f", "go_c", "go_p", "kegg"]
NGG = len(USED_GRAPHS)
N = 10000
NP = 10240
E = 320000
NC, NS = 2, 16
NW = NC * NS
DEC_BLOCK = 1024

_MESH = plsc.VectorSubcoreMesh(core_axis_name="c", subcore_axis_name="s",
                               num_cores=NC, num_subcores=NS)
_SC_PARAMS = pltpu.CompilerParams(use_tc_tiling_on_sc=False)


def _worker_id():
    return lax.axis_index("s") * NC + lax.axis_index("c")


L1_C = 80
L1_EPW = E // NW
L1_NCH = L1_EPW // L1_C
L1_RPT = NP // NS


def _mp1_body(t_ref, src_ref, dst_ref, out_ref, acc, idx_s, idx_d, rows, sem):
    cid = lax.axis_index("c")
    sid = lax.axis_index("s")
    w = _worker_id()
    r0 = sid * L1_RPT
    for g in range(NGG):
        pltpu.sync_copy(t_ref.at[pl.ds(g * NP + r0, L1_RPT)],
                        acc.at[pl.ds(r0, L1_RPT)])
        plsc.subcore_barrier()

        def body(j, _, g=g):
            eb = g * E + w * L1_EPW + j * L1_C
            pltpu.sync_copy(src_ref.at[pl.ds(eb, L1_C)], idx_s)
            pltpu.async_copy(t_ref.at[idx_s], rows, sem).wait()
            pltpu.sync_copy(dst_ref.at[pl.ds(eb, L1_C)], idx_d)
            pltpu.sync_copy(rows, acc.at[idx_d], add=True)
            return _

        lax.fori_loop(0, L1_NCH, body, 0)
        plsc.subcore_barrier()
        pltpu.sync_copy(acc.at[pl.ds(r0, L1_RPT)],
                        out_ref.at[g, cid, pl.ds(r0, L1_RPT)])
        plsc.subcore_barrier()


def _mp1_call(t_stack, src_glob, dst_raw):
    f = pl.kernel(
        _mp1_body,
        compiler_params=_SC_PARAMS,
        out_type=jax.ShapeDtypeStruct((NGG, NC, NP, 128), jnp.float32),
        mesh=_MESH,
        scratch_types=[
            pltpu.VMEM_SHARED((NP, 128), jnp.float32),
            pltpu.VMEM((L1_C,), jnp.int32),
            pltpu.VMEM((L1_C,), jnp.int32),
            pltpu.VMEM((L1_C, 128), jnp.float32),
            pltpu.SemaphoreType.DMA,
        ],
    )
    return f(t_stack, src_glob, dst_raw)


L2_C = 112
L2_EPW = NGG * E // NW
L2_NCH = L2_EPW // L2_C
L2_RPT = NGG * NP // NS


def _mp2_body(t_ref, src_ref, dst_ref, out_ref, acc, idx_s, idx_d, rows, sem):
    cid = lax.axis_index("c")
    sid = lax.axis_index("s")
    w = _worker_id()
    r0 = sid * L2_RPT
    pltpu.sync_copy(t_ref.at[pl.ds(r0, L2_RPT)], acc.at[pl.ds(r0, L2_RPT)])
    plsc.subcore_barrier()

    def body(j, _):
        eb = w * L2_EPW + j * L2_C
        pltpu.sync_copy(src_ref.at[pl.ds(eb, L2_C)], idx_s)
        pltpu.async_copy(t_ref.at[idx_s], rows, sem).wait()
        pltpu.sync_copy(dst_ref.at[pl.ds(eb, L2_C)], idx_d)
        pltpu.sync_copy(rows, acc.at[idx_d], add=True)
        return _

    lax.fori_loop(0, L2_NCH, body, 0)
    plsc.subcore_barrier()
    pltpu.sync_copy(acc.at[pl.ds(r0, L2_RPT)],
                    out_ref.at[cid, pl.ds(r0, L2_RPT)])


def _mp2_call(t2_stack, src_glob, dst_glob):
    f = pl.kernel(
        _mp2_body,
        compiler_params=_SC_PARAMS,
        out_type=jax.ShapeDtypeStruct((NC, NGG * NP, 16), jnp.float32),
        mesh=_MESH,
        scratch_types=[
            pltpu.VMEM_SHARED((NGG * NP, 16), jnp.float32),
            pltpu.VMEM((L2_C,), jnp.int32),
            pltpu.VMEM((L2_C,), jnp.int32),
            pltpu.VMEM((L2_C, 16), jnp.float32),
            pltpu.SemaphoreType.DMA,
        ],
    )
    return f(t2_stack, src_glob, dst_glob)


EDEC = 2 * E
DC_C = 80
DC_EPW = EDEC // NW
DC_NCH = DC_EPW // DC_C


def _dec_body(z_ref, u_ref, v_ref, out_ref, idx_u, idx_v, rows, sem):
    w = _worker_id()

    def body(j, _):
        eb = w * DC_EPW + j * DC_C
        pltpu.sync_copy(u_ref.at[pl.ds(eb, DC_C)], idx_u)
        pltpu.async_copy(z_ref.at[idx_u], rows, sem).wait()
        pltpu.sync_copy(v_ref.at[pl.ds(eb, DC_C)], idx_v)
        pltpu.async_copy(z_ref.at[idx_v], rows, sem, add=True).wait()
        pltpu.sync_copy(rows, out_ref.at[pl.ds(eb, DC_C)])
        return _

    lax.fori_loop(0, DC_NCH, body, 0)


def _dec_call(zcat, u, v):
    f = pl.kernel(
        _dec_body,
        compiler_params=_SC_PARAMS,
        out_type=jax.ShapeDtypeStruct((EDEC, 112), jnp.float32),
        mesh=_MESH,
        scratch_types=[
            pltpu.VMEM((DC_C,), jnp.int32),
            pltpu.VMEM((DC_C,), jnp.int32),
            pltpu.VMEM((DC_C, 112), jnp.float32),
            pltpu.SemaphoreType.DMA,
        ],
    )
    return f(zcat, u, v)


def _decode_mlp_body(emb_ref, wg_ref, bg_ref, ex_ref, l1w_ref, l1b_ref,
                     l2w_ref, l2b_ref, l3w_ref, l3b_ref, out_ref):
    emb = emb_ref[...]
    wv = jnp.dot(emb, wg_ref[...], preferred_element_type=jnp.float32) + bg_ref[...]
    wvx = jnp.dot(wv, ex_ref[...], preferred_element_type=jnp.float32)
    feats = emb * wvx
    h = jnp.maximum(jnp.dot(feats, l1w_ref[...], preferred_element_type=jnp.float32)
                    + l1b_ref[...], 0.0)
    h = jnp.maximum(jnp.dot(h, l2w_ref[...], preferred_element_type=jnp.float32)
                    + l2b_ref[...], 0.0)
    o = jnp.dot(h, l3w_ref[...], preferred_element_type=jnp.float32) + l3b_ref[...]
    out_ref[...] = 1.0 / (1.0 + jnp.exp(-o))


def _decode_mlp(emb, params):
    ne = emb.shape[0]
    d = NGG * 16
    wg = jnp.zeros((d, NGG), jnp.float32)
    for i, g in enumerate(USED_GRAPHS):
        wg = wg.at[16 * i:16 * (i + 1), i].set(params["w_" + g + "_W"][:, 0])
    bg = jnp.stack([params["w_" + g + "_b"][0] for g in USED_GRAPHS])[None, :]
    ex = jnp.repeat(jnp.eye(NGG, dtype=jnp.float32), 16, axis=1)

    grid = ne // DEC_BLOCK
    full = lambda shape: pl.BlockSpec(shape, lambda i: (0, 0))
    out = pl.pallas_call(
        _decode_mlp_body,
        grid=(grid,),
        in_specs=[
            pl.BlockSpec((DEC_BLOCK, d), lambda i: (i, 0)),
            full(wg.shape), full(bg.shape), full(ex.shape),
            full(params["L1_W"].shape), full((1, 32)),
            full(params["L2_W"].shape), full((1, 16)),
            full(params["L3_W"].shape), full((1, 1)),
        ],
        out_specs=pl.BlockSpec((DEC_BLOCK, 1), lambda i: (i, 0)),
        out_shape=jax.ShapeDtypeStruct((ne, 1), jnp.float32),
    )(emb, wg, bg, ex,
      params["L1_W"], params["L1_b"][None, :],
      params["L2_W"], params["L2_b"][None, :],
      params["L3_W"], params["L3_b"][None, :])
    return out[:, 0]


def kernel(x, sl_pos, sl_neg, kg_ppi, kg_reactome, kg_corum, kg_go_f,
           kg_go_c, kg_go_p, kg_kegg, params):
    edge_map = {"sl": sl_pos, "ppi": kg_ppi, "reactome": kg_reactome,
                "go_f": kg_go_f, "go_c": kg_go_c, "go_p": kg_go_p,
                "kegg": kg_kegg}
    eis = [edge_map[g] for g in USED_GRAPHS]
    src_glob = jnp.concatenate([ei[0] + g * NP for g, ei in enumerate(eis)])
    dst_raw = jnp.concatenate([ei[1] for ei in eis])
    dst_glob = jnp.concatenate([ei[1] + g * NP for g, ei in enumerate(eis)])

    deg = jax.ops.segment_sum(jnp.ones((NGG * E,), jnp.float32), dst_glob,
                              num_segments=NGG * NP) + 1.0
    dinv = (deg ** -0.5)[:, None]

    pad = jnp.zeros((NP - N, 128), jnp.float32)
    t1 = jnp.concatenate(
        [jnp.concatenate([x @ params[g + "_W1"], pad])
         for g in USED_GRAPHS], axis=0) * dinv

    p1 = _mp1_call(t1, src_glob, dst_raw)
    s1 = (p1[:, 0] + p1[:, 1]).reshape(NGG * NP, 128) - t1
    b1 = jnp.concatenate(
        [jnp.broadcast_to(params[g + "_b1"], (NP, 128)) for g in USED_GRAPHS])
    h = jax.nn.relu(dinv * s1 + b1)

    t2 = jnp.concatenate(
        [h[g * NP:(g + 1) * NP] @ params[USED_GRAPHS[g] + "_W2"]
         for g in range(NGG)], axis=0) * dinv

    p2 = _mp2_call(t2, src_glob, dst_glob)
    b2 = jnp.concatenate(
        [jnp.broadcast_to(params[g + "_b2"], (NP, 16)) for g in USED_GRAPHS])
    z = dinv * (p2[0] + p2[1] - t2) + b2
    zcat = z.reshape(NGG, NP, 16)[:, :N].transpose(1, 0, 2).reshape(N, NGG * 16)

    u = jnp.concatenate([sl_pos[0], sl_neg[0]])
    v = jnp.concatenate([sl_pos[1], sl_neg[1]])
    emb = _dec_call(zcat, u, v)
    return _decode_mlp(emb, params)


# speedup vs baseline: 15.1645x; 1.4823x over previous
import functools

import jax
import jax.numpy as jnp
from jax import lax
from jax.experimental import pallas as pl
from jax.experimental.pallas import tpu as pltpu
from jax.experimental.pallas import tpu_sc as plsc

USED_GRAPHS = ["sl", "ppi", "reactome", "go_f", "go_c", "go_p", "kegg"]
NGG = len(USED_GRAPHS)
N = 10000
NP = 10240
E = 320000
NC, NS = 2, 16
NW = NC * NS
DEC_BLOCK = 3200

_MESH = plsc.VectorSubcoreMesh(core_axis_name="c", subcore_axis_name="s",
                               num_cores=NC, num_subcores=NS)
_SC_PARAMS = pltpu.CompilerParams(use_tc_tiling_on_sc=False)


def _worker_id():
    return lax.axis_index("s") * NC + lax.axis_index("c")


L1_C = 80
L1_EPW = E // NW
L1_NCH = L1_EPW // L1_C
L1_RPT = NP // NS


def _mp1_body(t_ref, src_ref, dst_ref, out_ref, acc, idx_s, idx_d, rows, sem):
    cid = lax.axis_index("c")
    sid = lax.axis_index("s")
    w = _worker_id()
    r0 = sid * L1_RPT
    for g in range(NGG):
        pltpu.sync_copy(t_ref.at[pl.ds(g * NP + r0, L1_RPT)],
                        acc.at[pl.ds(r0, L1_RPT)])
        plsc.subcore_barrier()

        def body(j, _, g=g):
            eb = g * E + w * L1_EPW + j * L1_C
            pltpu.sync_copy(src_ref.at[pl.ds(eb, L1_C)], idx_s)
            pltpu.async_copy(t_ref.at[idx_s], rows, sem).wait()
            pltpu.sync_copy(dst_ref.at[pl.ds(eb, L1_C)], idx_d)
            pltpu.sync_copy(rows, acc.at[idx_d], add=True)
            return _

        lax.fori_loop(0, L1_NCH, body, 0)
        plsc.subcore_barrier()
        pltpu.sync_copy(acc.at[pl.ds(r0, L1_RPT)],
                        out_ref.at[g, cid, pl.ds(r0, L1_RPT)])
        plsc.subcore_barrier()


def _mp1_call(t_stack, src_glob, dst_raw):
    f = pl.kernel(
        _mp1_body,
        compiler_params=_SC_PARAMS,
        out_type=jax.ShapeDtypeStruct((NGG, NC, NP, 128), jnp.float32),
        mesh=_MESH,
        scratch_types=[
            pltpu.VMEM_SHARED((NP, 128), jnp.float32),
            pltpu.VMEM((L1_C,), jnp.int32),
            pltpu.VMEM((L1_C,), jnp.int32),
            pltpu.VMEM((L1_C, 128), jnp.float32),
            pltpu.SemaphoreType.DMA,
        ],
    )
    return f(t_stack, src_glob, dst_raw)


L2_C = 112
L2_EPW = NGG * E // NW
L2_NCH = L2_EPW // L2_C
L2_RPT = NGG * NP // NS


def _mp2_body(t_ref, src_ref, dst_ref, out_ref, acc, idx_s, idx_d, rows, sem):
    cid = lax.axis_index("c")
    sid = lax.axis_index("s")
    w = _worker_id()
    r0 = sid * L2_RPT
    pltpu.sync_copy(t_ref.at[pl.ds(r0, L2_RPT)], acc.at[pl.ds(r0, L2_RPT)])
    plsc.subcore_barrier()

    def body(j, _):
        eb = w * L2_EPW + j * L2_C
        pltpu.sync_copy(src_ref.at[pl.ds(eb, L2_C)], idx_s)
        pltpu.async_copy(t_ref.at[idx_s], rows, sem).wait()
        pltpu.sync_copy(dst_ref.at[pl.ds(eb, L2_C)], idx_d)
        pltpu.sync_copy(rows, acc.at[idx_d], add=True)
        return _

    lax.fori_loop(0, L2_NCH, body, 0)
    plsc.subcore_barrier()
    pltpu.sync_copy(acc.at[pl.ds(r0, L2_RPT)],
                    out_ref.at[cid, pl.ds(r0, L2_RPT)])


def _mp2_call(t2_stack, src_glob, dst_glob):
    f = pl.kernel(
        _mp2_body,
        compiler_params=_SC_PARAMS,
        out_type=jax.ShapeDtypeStruct((NC, NGG * NP, 16), jnp.float32),
        mesh=_MESH,
        scratch_types=[
            pltpu.VMEM_SHARED((NGG * NP, 16), jnp.float32),
            pltpu.VMEM((L2_C,), jnp.int32),
            pltpu.VMEM((L2_C,), jnp.int32),
            pltpu.VMEM((L2_C, 16), jnp.float32),
            pltpu.SemaphoreType.DMA,
        ],
    )
    return f(t2_stack, src_glob, dst_glob)


EDEC = 2 * E
DC_C = 80
DC_EPW = EDEC // NW
DC_NCH = DC_EPW // DC_C


def _dec_body(z_ref, u_ref, v_ref, out_ref, idx_u, idx_v, rows, sem):
    w = _worker_id()

    def body(j, _):
        eb = w * DC_EPW + j * DC_C
        pltpu.sync_copy(u_ref.at[pl.ds(eb, DC_C)], idx_u)
        pltpu.async_copy(z_ref.at[idx_u], rows, sem).wait()
        pltpu.sync_copy(v_ref.at[pl.ds(eb, DC_C)], idx_v)
        pltpu.async_copy(z_ref.at[idx_v], rows, sem, add=True).wait()
        pltpu.sync_copy(rows, out_ref.at[pl.ds(eb, DC_C)])
        return _

    lax.fori_loop(0, DC_NCH, body, 0)


def _dec_call(zcat, u, v):
    f = pl.kernel(
        _dec_body,
        compiler_params=_SC_PARAMS,
        out_type=jax.ShapeDtypeStruct((EDEC, 112), jnp.float32),
        mesh=_MESH,
        scratch_types=[
            pltpu.VMEM((DC_C,), jnp.int32),
            pltpu.VMEM((DC_C,), jnp.int32),
            pltpu.VMEM((DC_C, 112), jnp.float32),
            pltpu.SemaphoreType.DMA,
        ],
    )
    return f(zcat, u, v)


DG_C = 112
DG_EPW = NGG * E // NW
DG_NCH = DG_EPW // DG_C
NGNP = NGG * NP
DG_RPT = NGNP // NS


def _deg_body(dst_ref, zeros_ref, out_ref, acc, idx, ones_v, sem):
    cid = lax.axis_index("c")
    sid = lax.axis_index("s")
    w = _worker_id()
    r0 = sid * DG_RPT
    pltpu.sync_copy(zeros_ref.at[pl.ds(r0, DG_RPT)], acc.at[pl.ds(r0, DG_RPT)])
    for i in range(DG_C // 16):
        ones_v[pl.ds(16 * i, 16)] = jnp.full((16,), 1.0, jnp.float32)
    plsc.subcore_barrier()

    def body(j, _):
        eb = w * DG_EPW + j * DG_C
        pltpu.sync_copy(dst_ref.at[pl.ds(eb, DG_C)], idx)
        pltpu.sync_copy(ones_v, acc.at[idx], add=True)
        return _

    lax.fori_loop(0, DG_NCH, body, 0)
    plsc.subcore_barrier()
    pltpu.sync_copy(acc.at[pl.ds(r0, DG_RPT)],
                    out_ref.at[cid, pl.ds(r0, DG_RPT)])


def _deg_call(dst_glob):
    f = pl.kernel(
        _deg_body,
        compiler_params=_SC_PARAMS,
        out_type=jax.ShapeDtypeStruct((NC, NGNP), jnp.float32),
        mesh=_MESH,
        scratch_types=[
            pltpu.VMEM_SHARED((NGNP,), jnp.float32),
            pltpu.VMEM((DG_C,), jnp.int32),
            pltpu.VMEM((DG_C,), jnp.float32),
            pltpu.SemaphoreType.DMA,
        ],
    )
    return f(dst_glob, jnp.zeros((NGNP,), jnp.float32))


def _decode_mlp_body(emb_ref, wg_ref, bg_ref, ex_ref, l1w_ref, l1b_ref,
                     l2w_ref, l2b_ref, l3w_ref, l3b_ref, out_ref):
    emb = emb_ref[...]
    wv = jnp.dot(emb, wg_ref[...], preferred_element_type=jnp.float32) + bg_ref[...]
    wvx = jnp.dot(wv, ex_ref[...], preferred_element_type=jnp.float32)
    feats = emb * wvx
    h = jnp.maximum(jnp.dot(feats, l1w_ref[...], preferred_element_type=jnp.float32)
                    + l1b_ref[...], 0.0)
    h = jnp.maximum(jnp.dot(h, l2w_ref[...], preferred_element_type=jnp.float32)
                    + l2b_ref[...], 0.0)
    o = jnp.dot(h, l3w_ref[...], preferred_element_type=jnp.float32) + l3b_ref[...]
    out_ref[...] = 1.0 / (1.0 + jnp.exp(-o))


def _decode_mlp(emb, params):
    ne = emb.shape[0]
    d = NGG * 16
    wg = jnp.zeros((d, NGG), jnp.float32)
    for i, g in enumerate(USED_GRAPHS):
        wg = wg.at[16 * i:16 * (i + 1), i].set(params["w_" + g + "_W"][:, 0])
    bg = jnp.stack([params["w_" + g + "_b"][0] for g in USED_GRAPHS])[None, :]
    ex = jnp.repeat(jnp.eye(NGG, dtype=jnp.float32), 16, axis=1)

    grid = ne // DEC_BLOCK
    full = lambda shape: pl.BlockSpec(shape, lambda i: (0, 0))
    out = pl.pallas_call(
        _decode_mlp_body,
        grid=(grid,),
        in_specs=[
            pl.BlockSpec((DEC_BLOCK, d), lambda i: (i, 0)),
            full(wg.shape), full(bg.shape), full(ex.shape),
            full(params["L1_W"].shape), full((1, 32)),
            full(params["L2_W"].shape), full((1, 16)),
            full(params["L3_W"].shape), full((1, 1)),
        ],
        out_specs=pl.BlockSpec((DEC_BLOCK, 1), lambda i: (i, 0)),
        out_shape=jax.ShapeDtypeStruct((ne, 1), jnp.float32),
    )(emb, wg, bg, ex,
      params["L1_W"], params["L1_b"][None, :],
      params["L2_W"], params["L2_b"][None, :],
      params["L3_W"], params["L3_b"][None, :])
    return out[:, 0]


def kernel(x, sl_pos, sl_neg, kg_ppi, kg_reactome, kg_corum, kg_go_f,
           kg_go_c, kg_go_p, kg_kegg, params):
    edge_map = {"sl": sl_pos, "ppi": kg_ppi, "reactome": kg_reactome,
                "go_f": kg_go_f, "go_c": kg_go_c, "go_p": kg_go_p,
                "kegg": kg_kegg}
    eis = [edge_map[g] for g in USED_GRAPHS]
    src_glob = jnp.concatenate([ei[0] + g * NP for g, ei in enumerate(eis)])
    dst_raw = jnp.concatenate([ei[1] for ei in eis])
    dst_glob = jnp.concatenate([ei[1] + g * NP for g, ei in enumerate(eis)])

    dp = _deg_call(dst_glob)
    deg = dp[0] + dp[1] + 1.0
    dinv = (deg ** -0.5)[:, None]

    pad = jnp.zeros((NP - N, 128), jnp.float32)
    t1 = jnp.concatenate(
        [jnp.concatenate([x @ params[g + "_W1"], pad])
         for g in USED_GRAPHS], axis=0) * dinv

    p1 = _mp1_call(t1, src_glob, dst_raw)
    s1 = (p1[:, 0] + p1[:, 1]).reshape(NGG * NP, 128) - t1
    b1 = jnp.concatenate(
        [jnp.broadcast_to(params[g + "_b1"], (NP, 128)) for g in USED_GRAPHS])
    h = jax.nn.relu(dinv * s1 + b1)

    t2 = jnp.concatenate(
        [h[g * NP:(g + 1) * NP] @ params[USED_GRAPHS[g] + "_W2"]
         for g in range(NGG)], axis=0) * dinv

    p2 = _mp2_call(t2, src_glob, dst_glob)
    b2 = jnp.concatenate(
        [jnp.broadcast_to(params[g + "_b2"], (NP, 16)) for g in USED_GRAPHS])
    z = dinv * (p2[0] + p2[1] - t2) + b2
    zcat = z.reshape(NGG, NP, 16)[:, :N].transpose(1, 0, 2).reshape(N, NGG * 16)

    u = jnp.concatenate([sl_pos[0], sl_neg[0]])
    v = jnp.concatenate([sl_pos[1], sl_neg[1]])
    emb = _dec_call(zcat, u, v)
    return _decode_mlp(emb, params)
